# baseline (speedup 1.0000x reference)
import functools

import jax
import jax.numpy as jnp
from jax import lax
from jax.experimental import pallas as pl
from jax.experimental.pallas import tpu as pltpu
from jax.experimental.pallas import tpu_sc as plsc

_D = 128
_CHUNK = 128
_NW = 32
_NBUF = 6
_DEPTH = 4


def _make_gather(n_rows):
    b_per_w = n_rows // _NW
    n_chunks = b_per_w // _CHUNK
    mesh = plsc.VectorSubcoreMesh(core_axis_name="c", subcore_axis_name="s")

    @functools.partial(
        pl.kernel,
        mesh=mesh,
        out_type=jax.ShapeDtypeStruct((n_rows, _D), jnp.float32),
        scratch_types=[
            pltpu.VMEM((n_chunks, _CHUNK), jnp.int32),
            pltpu.VMEM((_NBUF * _CHUNK, _D), jnp.float32),
            pltpu.SemaphoreType.DMA((_NBUF,)),
            pltpu.SemaphoreType.DMA((_NBUF,)),
        ],
    )
    def gather_kernel(table_hbm, idx_hbm, out_hbm, idx_v, rows_v, gsem, psem):
        cid = lax.axis_index("c")
        sid = lax.axis_index("s")
        wid = sid * 2 + cid
        base = wid * b_per_w
        pltpu.sync_copy(idx_hbm.at[pl.ds(wid * n_chunks, n_chunks)], idx_v)

        def gather_desc(j, b):
            return pltpu.make_async_copy(
                table_hbm.at[idx_v.at[j]],
                rows_v.at[pl.ds(b * _CHUNK, _CHUNK)],
                gsem.at[b],
            )

        def pair_put_desc(j, b):
            return pltpu.make_async_copy(
                rows_v.at[pl.ds(b * _CHUNK, 2 * _CHUNK)],
                out_hbm.at[pl.ds(base + j * _CHUNK, 2 * _CHUNK)],
                psem.at[b],
            )

        for b in range(_DEPTH):
            gather_desc(b, b).start()

        def body(g, carry):
            j = g * 2
            b = lax.rem(j, _NBUF)
            gather_desc(j, b).wait()
            gather_desc(j + 1, b + 1).wait()
            pair_put_desc(j, b).start()
            jn = j + _DEPTH

            @pl.when(jn < n_chunks)
            def _():
                bn = lax.rem(jn, _NBUF)

                @pl.when(jn >= _NBUF)
                def _():
                    pair_put_desc(jn - _NBUF, bn).wait()

                gather_desc(jn, bn).start()
                gather_desc(jn + 1, bn + 1).start()

            return carry

        lax.fori_loop(0, n_chunks // 2, body, 0)

        for t in range(_NBUF // 2):
            jo = n_chunks - _NBUF + 2 * t
            pair_put_desc(jo, jo % _NBUF).wait()

    return gather_kernel


def kernel(input_ids, embed_table):
    batch, hist = input_ids.shape
    idx = input_ids.astype(jnp.int32).T.reshape(-1, _CHUNK)
    out = _make_gather(batch * hist)(embed_table, idx)
    return out.reshape(hist, batch, _D).transpose(1, 0, 2)


# speedup vs baseline: 1.0010x; 1.0010x over previous
import functools

import jax
import jax.numpy as jnp
from jax import lax
from jax.experimental import pallas as pl
from jax.experimental.pallas import tpu as pltpu
from jax.experimental.pallas import tpu_sc as plsc

_D = 128
_CHUNK = 128
_NW = 32
_NBUF = 6
_DEPTH = 5


def _make_gather(n_rows):
    b_per_w = n_rows // _NW
    n_chunks = b_per_w // _CHUNK
    mesh = plsc.VectorSubcoreMesh(core_axis_name="c", subcore_axis_name="s")

    @functools.partial(
        pl.kernel,
        mesh=mesh,
        out_type=jax.ShapeDtypeStruct((n_rows, _D), jnp.float32),
        scratch_types=[
            pltpu.VMEM((n_chunks, _CHUNK), jnp.int32),
            pltpu.VMEM((_NBUF, _CHUNK, _D), jnp.float32),
            pltpu.SemaphoreType.DMA((_NBUF,)),
            pltpu.SemaphoreType.DMA((_NBUF,)),
        ],
    )
    def gather_kernel(table_hbm, idx_hbm, out_hbm, idx_v, rows_v, gsem, psem):
        cid = lax.axis_index("c")
        sid = lax.axis_index("s")
        wid = sid * 2 + cid
        base = wid * b_per_w
        pltpu.sync_copy(idx_hbm.at[pl.ds(wid * n_chunks, n_chunks)], idx_v)

        def gather_desc(j, b):
            return pltpu.make_async_copy(
                table_hbm.at[idx_v.at[j]], rows_v.at[b], gsem.at[b]
            )

        def put_desc(j, b):
            return pltpu.make_async_copy(
                rows_v.at[b],
                out_hbm.at[pl.ds(base + j * _CHUNK, _CHUNK)],
                psem.at[b],
            )

        for b in range(_DEPTH):
            gather_desc(b, b).start()

        def body(j, carry):
            b = lax.rem(j, _NBUF)
            gather_desc(j, b).wait()
            put_desc(j, b).start()
            jn = j + _DEPTH

            @pl.when(jn < n_chunks)
            def _():
                bn = lax.rem(jn, _NBUF)

                @pl.when(jn >= _NBUF)
                def _():
                    put_desc(jn - _NBUF, bn).wait()

                gather_desc(jn, bn).start()

            return carry

        lax.fori_loop(0, n_chunks, body, 0)

        for t in range(_NBUF):
            jo = n_chunks - _NBUF + t
            put_desc(jo, jo % _NBUF).wait()

    return gather_kernel


def kernel(input_ids, embed_table):
    batch, hist = input_ids.shape
    idx = input_ids.astype(jnp.int32).T.reshape(-1, _CHUNK)
    out = _make_gather(batch * hist)(embed_table, idx)
    return out.reshape(hist, batch, _D).transpose(1, 0, 2)
